# baseline (speedup 1.0000x reference)
import functools

import jax
import jax.numpy as jnp
import numpy as np
from jax.experimental import pallas as pl
from jax.experimental.pallas import tpu as pltpu

IMG = 32
KS = 5
H1 = IMG - KS + 1
P1 = H1 // 2
H2 = P1 - KS + 1
P2 = H2 // 2
NF = 16 * P2 * P2

B_TILE = 1024
GY1 = 4


@functools.lru_cache(maxsize=None)
def _const_selectors(cin):
    kg1 = (GY1 + 4) * IMG
    s1 = np.zeros((KS * KS, GY1, 2, P1, kg1), np.float32)
    for ky in range(KS):
        for kx in range(KS):
            for yl in range(GY1):
                for par in range(2):
                    for x1 in range(P1):
                        s1[ky * KS + kx, yl, par, x1,
                           (yl + ky) * IMG + 2 * x1 + par + kx] = 1.0
    s2 = np.zeros((6 * KS * KS, 2, 2, P2, 6 * 84), np.float32)
    for c in range(6):
        for ky in range(KS):
            for kx in range(KS):
                for yl in range(2):
                    for par in range(2):
                        for xh in range(P2):
                            s2[c * KS * KS + ky * KS + kx, yl, par, xh,
                               (yl + ky) * 84 + c * P1 + 2 * xh + par + kx] = 1.0
    return s1, s2


def _lenet_body(x_ref, a1_ref, bc1_ref, a2_ref, bc2_ref,
                wf1_ref, bf1_ref, wf2_ref, bf2_ref, wf3_ref, bf3_ref,
                out_ref, xg_s, c1_s, p1_s, c2_s, p2_s):
    f32 = jnp.float32
    bf16 = jnp.bfloat16
    cin = x_ref.shape[0] // (IMG * IMG)
    R_IMG = IMG * IMG
    kg1 = (GY1 + 4) * IMG
    KG = cin * kg1
    MG = GY1 * 168

    NCH = 1
    BC = B_TILE // NCH

    def conv1_group(g, s, hh):
        buf = ((g % 2) * NCH + hh) * KG
        for c in range(cin):
            base = c * R_IMG + g * GY1 * IMG
            xg_s[buf + c * kg1:buf + (c + 1) * kg1, s] = \
                x_ref[base:base + kg1, s].astype(bf16)
        r = jnp.dot(a1_ref[...], xg_s[buf:buf + KG, s],
                    preferred_element_type=f32)
        c1_s[g * MG:(g + 1) * MG, s] = jnp.maximum(
            r + bc1_ref[...], 0.0).astype(bf16)

    def pool1_row(y1, s):
        b0 = (2 * y1) * 6 * H1
        b1 = (2 * y1 + 1) * 6 * H1
        p1_s[y1 * 84:(y1 + 1) * 84, s] = 0.25 * (
            c1_s[b0:b0 + 84, s] + c1_s[b0 + 84:b0 + 168, s]
            + c1_s[b1:b1 + 84, s] + c1_s[b1 + 84:b1 + 168, s])

    def conv2_pair(q, s):
        r = jnp.dot(a2_ref[...], p1_s[q * 168:q * 168 + 504, s],
                    preferred_element_type=f32)
        c2_s[q * 320:(q + 1) * 320, s] = jnp.maximum(
            r + bc2_ref[...], 0.0).astype(bf16)
        b0 = q * 320
        p2_s[q * 80:(q + 1) * 80, s] = 0.25 * (
            c2_s[b0:b0 + 80, s] + c2_s[b0 + 80:b0 + 160, s]
            + c2_s[b0 + 160:b0 + 240, s] + c2_s[b0 + 240:b0 + 320, s])

    def fc_chain(s, s_out):
        h = jnp.dot(wf1_ref[...], p2_s[:, s], preferred_element_type=f32)
        h = jnp.maximum(h + bf1_ref[...], 0.0).astype(bf16)
        h = jnp.dot(wf2_ref[...], h, preferred_element_type=f32)
        h = jnp.maximum(h + bf2_ref[...], 0.0).astype(bf16)
        r = jnp.dot(wf3_ref[...], h, preferred_element_type=f32) + bf3_ref[...]
        out_ref[s_out, :] = r.T

    chains = [slice(hh * BC, (hh + 1) * BC) for hh in range(NCH)]
    for g in range(H1 // GY1):
        for hh, s in enumerate(chains):
            conv1_group(g, s, hh)
            for y1 in (2 * g, 2 * g + 1):
                pool1_row(y1, s)
            if g >= 3:
                conv2_pair(g - 3, s)
    for s in chains:
        conv2_pair(4, s)
        fc_chain(s, s)


def kernel(w1, b1, w2, b2, wf1, bf1, wf2, bf2, wf3, bf3, x):
    n, cin, h, w = x.shape
    if (h, w) != (IMG, IMG):
        raise ValueError("expects 32x32 inputs")
    f32 = jnp.float32
    x = x.astype(f32)
    nc = wf3.shape[0]
    n_pad = pl.cdiv(n, B_TILE) * B_TILE

    x2 = x.transpose(1, 2, 3, 0).reshape(cin * IMG * IMG, n)
    if n_pad != n:
        x2 = jnp.pad(x2, ((0, 0), (0, n_pad - n)))

    s1, s2 = _const_selectors(cin)
    bf16 = jnp.bfloat16
    w1coj = w1.reshape(6, cin, KS * KS).transpose(1, 0, 2).astype(bf16)
    a1 = jnp.einsum('coj,jypxk->ypoxck', w1coj,
                    jnp.asarray(s1, dtype=bf16),
                    preferred_element_type=bf16).reshape(
                        GY1 * 168, cin * (GY1 + 4) * IMG)
    a2 = jnp.einsum('oj,jypxk->ypoxk', w2.astype(bf16),
                    jnp.asarray(s2, dtype=bf16),
                    preferred_element_type=bf16).reshape(320, 6 * 84)
    wf1p = (wf1.reshape(120, 16, P2, P2).transpose(0, 2, 1, 3)
            .reshape(120, NF).astype(bf16))
    wf2 = wf2.astype(bf16)
    wf3 = wf3.astype(bf16)
    bc1 = jnp.broadcast_to(b1[None, None, :, None],
                           (GY1, 2, 6, P1)).reshape(GY1 * 168, 1)
    bc2 = jnp.broadcast_to(b2[None, None, :, None],
                           (2, 2, 16, P2)).reshape(320, 1)

    def vmem_full(a):
        return pl.BlockSpec(a.shape, lambda i: (0,) * a.ndim)

    out = pl.pallas_call(
        _lenet_body,
        out_shape=jax.ShapeDtypeStruct((n_pad, nc), f32),
        grid=(n_pad // B_TILE,),
        in_specs=[
            pl.BlockSpec((cin * IMG * IMG, B_TILE), lambda i: (0, i)),
            vmem_full(a1), vmem_full(bc1), vmem_full(a2), vmem_full(bc2),
            vmem_full(wf1p), vmem_full(bf1),
            vmem_full(wf2), vmem_full(bf2),
            vmem_full(wf3), vmem_full(bf3),
        ],
        out_specs=pl.BlockSpec((B_TILE, nc), lambda i: (i, 0)),
        scratch_shapes=[
            pltpu.VMEM((2 * cin * (GY1 + 4) * IMG,
                        B_TILE), bf16),
            pltpu.VMEM((6 * H1 * H1, B_TILE), bf16),
            pltpu.VMEM((6 * P1 * P1, B_TILE), bf16),
            pltpu.VMEM((16 * H2 * H2, B_TILE), bf16),
            pltpu.VMEM((NF, B_TILE), bf16),
        ],
        compiler_params=pltpu.CompilerParams(
            dimension_semantics=("parallel",)),
    )(x2, a1, bc1, a2, bc2, wf1p, bf1, wf2, bf2, wf3, bf3)
    return out[:n]


# speedup vs baseline: 1.1253x; 1.1253x over previous
import functools

import jax
import jax.numpy as jnp
import numpy as np
from jax.experimental import pallas as pl
from jax.experimental.pallas import tpu as pltpu

IMG = 32
KS = 5
H1 = IMG - KS + 1
P1 = H1 // 2
H2 = P1 - KS + 1
P2 = H2 // 2
NF = 16 * P2 * P2

B_TILE = 1024
GY1 = 4


@functools.lru_cache(maxsize=None)
def _const_selectors(cin):
    kg1 = (GY1 + 4) * IMG
    s1 = np.zeros((KS * KS, GY1, 2, P1, kg1), np.float32)
    for ky in range(KS):
        for kx in range(KS):
            for yl in range(GY1):
                for par in range(2):
                    for x1 in range(P1):
                        s1[ky * KS + kx, yl, par, x1,
                           (yl + ky) * IMG + 2 * x1 + par + kx] = 1.0
    s2 = np.zeros((6 * KS * KS, 2, 2, P2, 6 * 84), np.float32)
    for c in range(6):
        for ky in range(KS):
            for kx in range(KS):
                for yl in range(2):
                    for par in range(2):
                        for xh in range(P2):
                            s2[c * KS * KS + ky * KS + kx, yl, par, xh,
                               (yl + ky) * 84 + c * P1 + 2 * xh + par + kx] = 1.0
    return s1, s2


def _lenet_body(x_ref, a1_ref, a2_ref, wf1_ref, wf2_ref, wf3_ref, bias_ref,
                out_ref, xg_s, c1_s, p1_s, c2_s, p2_s):
    f32 = jnp.float32
    bf16 = jnp.bfloat16
    nc = out_ref.shape[1]
    bc1 = bias_ref[0:672, :]
    bc2 = bias_ref[672:992, :]
    bf1 = bias_ref[992:1112, :]
    bf2 = bias_ref[1112:1196, :]
    bf3 = bias_ref[1196:1196 + nc, :]
    cin = x_ref.shape[0] // (IMG * IMG)
    R_IMG = IMG * IMG
    kg1 = (GY1 + 4) * IMG
    KG = cin * kg1
    MG = GY1 * 168

    NCH = 1
    BC = B_TILE // NCH

    def conv1_group(g, s, hh):
        buf = ((g % 2) * NCH + hh) * KG
        for c in range(cin):
            base = c * R_IMG + g * GY1 * IMG
            xg_s[buf + c * kg1:buf + (c + 1) * kg1, s] = \
                x_ref[base:base + kg1, s].astype(bf16)
        r = jnp.dot(a1_ref[...], xg_s[buf:buf + KG, s],
                    preferred_element_type=f32)
        c1_s[g * MG:(g + 1) * MG, s] = jnp.maximum(
            r + bc1, 0.0).astype(bf16)

    def pool1_row(y1, s):
        b0 = (2 * y1) * 6 * H1
        b1 = (2 * y1 + 1) * 6 * H1
        p1_s[y1 * 84:(y1 + 1) * 84, s] = 0.25 * (
            c1_s[b0:b0 + 84, s] + c1_s[b0 + 84:b0 + 168, s]
            + c1_s[b1:b1 + 84, s] + c1_s[b1 + 84:b1 + 168, s])

    def conv2_pair(q, s):
        r = jnp.dot(a2_ref[...], p1_s[q * 168:q * 168 + 504, s],
                    preferred_element_type=f32)
        c2_s[q * 320:(q + 1) * 320, s] = jnp.maximum(
            r + bc2, 0.0).astype(bf16)
        b0 = q * 320
        p2_s[q * 80:(q + 1) * 80, s] = 0.25 * (
            c2_s[b0:b0 + 80, s] + c2_s[b0 + 80:b0 + 160, s]
            + c2_s[b0 + 160:b0 + 240, s] + c2_s[b0 + 240:b0 + 320, s])

    def fc_chain(s, s_out):
        h = jnp.dot(wf1_ref[...].astype(bf16), p2_s[:, s],
                    preferred_element_type=f32)
        h = jnp.maximum(h + bf1, 0.0).astype(bf16)
        h = jnp.dot(wf2_ref[...].astype(bf16), h, preferred_element_type=f32)
        h = jnp.maximum(h + bf2, 0.0).astype(bf16)
        r = jnp.dot(wf3_ref[...].astype(bf16), h,
                    preferred_element_type=f32) + bf3
        out_ref[s_out, :] = r.T

    chains = [slice(hh * BC, (hh + 1) * BC) for hh in range(NCH)]
    for g in range(H1 // GY1):
        for hh, s in enumerate(chains):
            conv1_group(g, s, hh)
            for y1 in (2 * g, 2 * g + 1):
                pool1_row(y1, s)
            if g >= 3:
                conv2_pair(g - 3, s)
    for s in chains:
        conv2_pair(4, s)
        fc_chain(s, s)


def kernel(w1, b1, w2, b2, wf1, bf1, wf2, bf2, wf3, bf3, x):
    n, cin, h, w = x.shape
    if (h, w) != (IMG, IMG):
        raise ValueError("expects 32x32 inputs")
    f32 = jnp.float32
    x = x.astype(f32)
    nc = wf3.shape[0]
    n_pad = pl.cdiv(n, B_TILE) * B_TILE

    x2 = x.transpose(1, 2, 3, 0).reshape(cin * IMG * IMG, n)
    if n_pad != n:
        x2 = jnp.pad(x2, ((0, 0), (0, n_pad - n)))

    s1, s2 = _const_selectors(cin)
    bf16 = jnp.bfloat16
    w1coj = w1.reshape(6, cin, KS * KS).transpose(1, 0, 2).astype(bf16)
    a1 = jnp.einsum('coj,jypxk->ypoxck', w1coj,
                    jnp.asarray(s1, dtype=bf16),
                    preferred_element_type=bf16).reshape(
                        GY1 * 168, cin * (GY1 + 4) * IMG)
    a2 = jnp.einsum('oj,jypxk->ypoxk', w2.astype(bf16),
                    jnp.asarray(s2, dtype=bf16),
                    preferred_element_type=bf16).reshape(320, 6 * 84)
    wf1p = (wf1.reshape(120, 16, P2, P2).transpose(0, 2, 1, 3)
            .reshape(120, NF))
    biases = jnp.concatenate([
        jnp.broadcast_to(b1[None, None, :, None],
                         (GY1, 2, 6, P1)).reshape(GY1 * 168, 1),
        jnp.broadcast_to(b2[None, None, :, None],
                         (2, 2, 16, P2)).reshape(320, 1),
        bf1, bf2, bf3], axis=0)

    def vmem_full(a):
        return pl.BlockSpec(a.shape, lambda i: (0,) * a.ndim)

    out = pl.pallas_call(
        _lenet_body,
        out_shape=jax.ShapeDtypeStruct((n_pad, nc), f32),
        grid=(n_pad // B_TILE,),
        in_specs=[
            pl.BlockSpec((cin * IMG * IMG, B_TILE), lambda i: (0, i)),
            vmem_full(a1), vmem_full(a2), vmem_full(wf1p),
            vmem_full(wf2), vmem_full(wf3), vmem_full(biases),
        ],
        out_specs=pl.BlockSpec((B_TILE, nc), lambda i: (i, 0)),
        scratch_shapes=[
            pltpu.VMEM((2 * cin * (GY1 + 4) * IMG,
                        B_TILE), bf16),
            pltpu.VMEM((6 * H1 * H1, B_TILE), bf16),
            pltpu.VMEM((6 * P1 * P1, B_TILE), bf16),
            pltpu.VMEM((16 * H2 * H2, B_TILE), bf16),
            pltpu.VMEM((NF, B_TILE), bf16),
        ],
        compiler_params=pltpu.CompilerParams(
            dimension_semantics=("parallel",)),
    )(x2, a1, a2, wf1p, wf2, wf3, biases)
    return out[:n]
